# baseline (speedup 1.0000x reference)
import jax
import jax.numpy as jnp
from jax.experimental import pallas as pl
from jax.experimental.pallas import tpu as pltpu

_BB = 4
_HB = 2


def _add_kernel(x1_ref, x2_ref, pos_ref, o_ref):
    pos = pos_ref[...][None, :, :]
    o_ref[0:_HB] = x1_ref[...] + pos
    o_ref[_HB:_BB] = x2_ref[...] + pos


def kernel(input_patch, pos_table):
    B, P, D = input_patch.shape
    grid = (B // _BB,)
    return pl.pallas_call(
        _add_kernel,
        grid=grid,
        in_specs=[
            pl.BlockSpec((_HB, P, D), lambda i: (2 * i, 0, 0)),
            pl.BlockSpec((_HB, P, D), lambda i: (2 * i + 1, 0, 0)),
            pl.BlockSpec((P, D), lambda i: (0, 0)),
        ],
        out_specs=pl.BlockSpec((_BB, P, D), lambda i: (i, 0, 0)),
        out_shape=jax.ShapeDtypeStruct((B, P, D), input_patch.dtype),
        compiler_params=pltpu.CompilerParams(dimension_semantics=("parallel",)),
    )(input_patch, input_patch, pos_table)
